# baseline (speedup 1.0000x reference)
import jax
import jax.numpy as jnp
import numpy as np
from jax.experimental import pallas as pl

HIDDEN = 1024
KDIM = 256
KNUM = 256
VDIM = 1024
KNN = 16
HEAD = 2


def _layer_norm(x, eps=1e-5):
    m = jnp.mean(x, axis=-1, keepdims=True)
    v = jnp.var(x, axis=-1, keepdims=True)
    return (x - m) / jnp.sqrt(v + eps)


def kernel(hidden_state, Wq, keys, Wv, Wsw, values_for_look_up):
    prefix = hidden_state.shape[:-1]
    bs = int(np.prod(prefix))
    x = hidden_state.reshape(bs, HIDDEN)
    q = x @ Wq.T
    q = q.reshape(bs, HEAD, 2, KDIM)
    q = _layer_norm(q)
    q = q.reshape(bs, HEAD, 2 * KDIM)
    k1 = keys[:, 0]
    k2 = keys[:, 1]
    q1 = q[:, :, :KDIM]
    q2 = q[:, :, KDIM:]
    s1 = jnp.einsum('blh,lkh->blk', q1, k1)
    s2 = jnp.einsum('blh,lkh->blk', q2, k2)
    s1, i1 = jax.lax.top_k(s1, KNN)
    s2, i2 = jax.lax.top_k(s2, KNN)
    all_scores = (s1[:, :, :, None] + s2[:, :, None, :]).reshape(bs, HEAD, KNN * KNN)
    all_indices = (i1[:, :, :, None] * KNUM + i2[:, :, None, :]).reshape(bs, HEAD, KNN * KNN)
    scores, best = jax.lax.top_k(all_scores, KNN)
    indices = jnp.take_along_axis(all_indices, best, axis=2)
    scores = jax.nn.softmax(scores.astype(jnp.float32), axis=-1)
    indices = indices.reshape(bs, HEAD * KNN)
    scores = scores.reshape(bs, HEAD * KNN)
    vals = jnp.take(values_for_look_up, indices, axis=0)
    mem_out = jnp.sum(vals * scores[:, :, None], axis=1)
    gate = jax.nn.silu(x @ Wsw.T)
    out = (mem_out * gate) @ Wv.T
    return out.reshape(prefix + (HIDDEN,))


# speedup vs baseline: 1.6324x; 1.6324x over previous
import functools

import jax
import jax.numpy as jnp
import numpy as np
from jax import lax
from jax.experimental import pallas as pl
from jax.experimental.pallas import tpu as pltpu
from jax.experimental.pallas import tpu_sc as plsc

HIDDEN = 1024
KDIM = 256
KNUM = 256
VDIM = 1024
KNN = 16
HEAD = 2

BS = 2048
TILE = 256
INT_MIN = np.int32(-2147483648)
MASK_LOW = np.int32(-256)
LOW = np.int32(255)

_E1_np = np.zeros((KNN, KNN * KNN), np.float32)
_E2_np = np.zeros((KNN, KNN * KNN), np.float32)
for _a in range(KNN):
    for _b in range(KNN):
        _E1_np[_a, _a * KNN + _b] = 1.0
        _E2_np[_b, _a * KNN + _b] = 1.0
_R_np = np.zeros((HEAD * KNN, HEAD * KNN * 16), np.float32)
for _j in range(HEAD * KNN):
    _R_np[_j, _j * 16:(_j + 1) * 16] = 1.0


NEG_INF = np.float32(-np.inf)
BIG_I32 = np.int32(1 << 30)


def _topk16(s, lane):
    vals, lanes = [], []
    for _ in range(KNN):
        mx = jnp.max(s, axis=1, keepdims=True)
        pick = jnp.min(jnp.where(s == mx, lane, BIG_I32), axis=1, keepdims=True)
        s = jnp.where(lane == pick, NEG_INF, s)
        vals.append(mx)
        lanes.append(pick)
    return vals, lanes


def _score_kernel(x_ref, wq_ref, keys_ref, e1_ref, e2_ref, r_ref,
                  idx_ref, wrep_ref):
    t = x_ref.shape[0]
    x = x_ref[...]
    q = lax.dot_general(x, wq_ref[...], (((1,), (1,)), ((), ())),
                        preferred_element_type=jnp.float32)
    lane = lax.broadcasted_iota(jnp.int32, (t, KNUM), 1)
    iota16 = lax.broadcasted_iota(jnp.int32, (t, KNN), 1)
    w_heads = []
    for h in range(HEAD):
        svs, ifs = [], []
        for half in range(2):
            c = h * 2 + half
            qc = q[:, c * KDIM:(c + 1) * KDIM]
            m = jnp.mean(qc, axis=1, keepdims=True)
            d = qc - m
            v = jnp.mean(d * d, axis=1, keepdims=True)
            qn = d * lax.rsqrt(v + 1e-5)
            s = lax.dot_general(qn, keys_ref[c], (((1,), (1,)), ((), ())),
                                preferred_element_type=jnp.float32)
            vals, lanes = _topk16(s, lane)
            svs.append(jnp.concatenate(vals, axis=1))
            ifs.append(jnp.concatenate(lanes, axis=1).astype(jnp.float32))
        cand = (jnp.dot(svs[0], e1_ref[...], preferred_element_type=jnp.float32, precision=lax.Precision.HIGHEST)
                + jnp.dot(svs[1], e2_ref[...], preferred_element_type=jnp.float32, precision=lax.Precision.HIGHEST))
        sc_cols, vi_cols = [], []
        sref = cand
        for _ in range(KNN):
            mx = jnp.max(sref, axis=1, keepdims=True)
            j = jnp.min(jnp.where(sref == mx, lane, BIG_I32), axis=1, keepdims=True)
            sref = jnp.where(lane == j, NEG_INF, sref)
            a = lax.shift_right_arithmetic(j, 4)
            b = j & np.int32(15)
            i1k = jnp.sum(jnp.where(iota16 == a, ifs[0], 0.0), axis=1, keepdims=True)
            i2k = jnp.sum(jnp.where(iota16 == b, ifs[1], 0.0), axis=1, keepdims=True)
            vi_cols.append(i1k * np.float32(KNUM) + i2k)
            sc_cols.append(mx)
        sc = jnp.concatenate(sc_cols, axis=1)
        vidx = jnp.concatenate(vi_cols, axis=1)
        e = jnp.exp(sc - jnp.max(sc, axis=1, keepdims=True))
        w = e / jnp.sum(e, axis=1, keepdims=True)
        idx_ref[:, h * KNN:(h + 1) * KNN] = vidx.astype(jnp.int32)
        w_heads.append(w)
    wfull = jnp.concatenate(w_heads, axis=1)
    wrep_ref[...] = jnp.dot(wfull, r_ref[...],
                            preferred_element_type=jnp.float32, precision=lax.Precision.HIGHEST)


def _out_kernel(x_ref, mem_ref, wsw_ref, wv_ref, o_ref):
    x = x_ref[...]
    g = lax.dot_general(x, wsw_ref[...], (((1,), (1,)), ((), ())),
                        preferred_element_type=jnp.float32)
    gate = g / (1.0 + jnp.exp(-g))
    hgate = mem_ref[...] * gate
    o_ref[...] = lax.dot_general(hgate, wv_ref[...], (((1,), (1,)), ((), ())),
                                 preferred_element_type=jnp.float32)


def _scores_tc(x, Wq, keys4, e1, e2, rrep):
    grid = (BS // TILE,)
    return pl.pallas_call(
        _score_kernel,
        grid=grid,
        in_specs=[
            pl.BlockSpec((TILE, HIDDEN), lambda i: (i, 0)),
            pl.BlockSpec((KDIM * 2 * HEAD, HIDDEN), lambda i: (0, 0)),
            pl.BlockSpec((4, KNUM, KDIM), lambda i: (0, 0, 0)),
            pl.BlockSpec((KNN, KNN * KNN), lambda i: (0, 0)),
            pl.BlockSpec((KNN, KNN * KNN), lambda i: (0, 0)),
            pl.BlockSpec((HEAD * KNN, HEAD * KNN * 16), lambda i: (0, 0)),
        ],
        out_specs=[
            pl.BlockSpec((TILE, HEAD * KNN), lambda i: (i, 0)),
            pl.BlockSpec((TILE, HEAD * KNN * 16), lambda i: (i, 0)),
        ],
        out_shape=[
            jax.ShapeDtypeStruct((BS, HEAD * KNN), jnp.int32),
            jax.ShapeDtypeStruct((BS, HEAD * KNN * 16), jnp.float32),
        ],
    )(x, Wq, keys4, e1, e2, rrep)


def _output_tc(x, mem_out, Wsw, Wv):
    grid = (BS // TILE,)
    return pl.pallas_call(
        _out_kernel,
        grid=grid,
        in_specs=[
            pl.BlockSpec((TILE, HIDDEN), lambda i: (i, 0)),
            pl.BlockSpec((TILE, VDIM), lambda i: (i, 0)),
            pl.BlockSpec((VDIM, HIDDEN), lambda i: (0, 0)),
            pl.BlockSpec((HIDDEN, VDIM), lambda i: (0, 0)),
        ],
        out_specs=pl.BlockSpec((TILE, HIDDEN), lambda i: (i, 0)),
        out_shape=jax.ShapeDtypeStruct((BS, HIDDEN), jnp.float32),
    )(x, mem_out, Wsw, Wv)


_NC, _NS = 2, 16
_NW = _NC * _NS
_TPW = BS // _NW
_K = HEAD * KNN
_NG = VDIM // 16


def _sc_gather(indices, wrep, values):
    mesh = plsc.VectorSubcoreMesh(core_axis_name="c", subcore_axis_name="s")

    @functools.partial(
        pl.kernel,
        mesh=mesh,
        out_type=jax.ShapeDtypeStruct((BS, VDIM), jnp.float32),
        scratch_types=[
            pltpu.VMEM((_TPW, _K), jnp.int32),
            pltpu.VMEM((_TPW, _K * 16), jnp.float32),
            pltpu.VMEM((2, _K, VDIM), jnp.float32),
            pltpu.VMEM((VDIM,), jnp.float32),
            pltpu.SemaphoreType.DMA,
            pltpu.SemaphoreType.DMA,
        ],
    )
    def gather_kernel(idx_hbm, w_hbm, values_hbm, out_hbm,
                      idx_v, w_v, rows_v, acc_v, sem0, sem1):
        wid = lax.axis_index("s") * _NC + lax.axis_index("c")
        base = wid * _TPW
        pltpu.sync_copy(idx_hbm.at[pl.ds(base, _TPW)], idx_v)
        pltpu.sync_copy(w_hbm.at[pl.ds(base, _TPW)], w_v)

        def start_gather(tok, buf, sem):
            pltpu.async_copy(values_hbm.at[idx_v.at[tok]], rows_v.at[buf], sem)

        def wait(src_tok, buf, sem):
            pltpu.make_async_copy(values_hbm.at[idx_v.at[src_tok]],
                                  rows_v.at[buf], sem).wait()

        def compute(tok, buf):
            w_regs = [w_v[tok, pl.ds(r * 16, 16)] for r in range(_K)]

            def gbody(g, carry):
                o = g * 16
                acc = rows_v[buf, 0, pl.ds(o, 16)] * w_regs[0]
                for r in range(1, _K):
                    acc = acc + rows_v[buf, r, pl.ds(o, 16)] * w_regs[r]
                acc_v[pl.ds(o, 16)] = acc
                return carry

            lax.fori_loop(0, _NG, gbody, 0, unroll=2)
            pltpu.sync_copy(acc_v, out_hbm.at[base + tok])

        start_gather(0, 0, sem0)

        def body(t2, carry):
            t0 = t2 * 2
            t1 = t0 + 1
            start_gather(t1, 1, sem1)
            wait(t0, 0, sem0)
            compute(t0, 0)

            @pl.when(t2 < _TPW // 2 - 1)
            def _():
                start_gather(t0 + 2, 0, sem0)

            wait(t1, 1, sem1)
            compute(t1, 1)
            return carry

        lax.fori_loop(0, _TPW // 2, body, 0)

    return gather_kernel(indices, wrep, values)


def kernel(hidden_state, Wq, keys, Wv, Wsw, values_for_look_up):
    prefix = hidden_state.shape[:-1]
    x = hidden_state.reshape(BS, HIDDEN)
    keys4 = keys.reshape(HEAD * 2, KNUM, KDIM)
    e1 = jnp.asarray(_E1_np)
    e2 = jnp.asarray(_E2_np)
    indices, wrep = _scores_tc(x, Wq, keys4, e1, e2, jnp.asarray(_R_np))
    mem_out = _sc_gather(indices, wrep, values_for_look_up)
    out = _output_tc(x, mem_out, Wsw, Wv)
    return out.reshape(prefix + (HIDDEN,))


# speedup vs baseline: 1.9674x; 1.2052x over previous
import functools

import jax
import jax.numpy as jnp
import numpy as np
from jax import lax
from jax.experimental import pallas as pl
from jax.experimental.pallas import tpu as pltpu
from jax.experimental.pallas import tpu_sc as plsc

HIDDEN = 1024
KDIM = 256
KNUM = 256
VDIM = 1024
KNN = 16
HEAD = 2

BS = 2048
TILE = 256
INT_MIN = np.int32(-2147483648)
MASK_LOW = np.int32(-256)
LOW = np.int32(255)

_E1_np = np.zeros((KNN, KNN * KNN), np.float32)
_E2_np = np.zeros((KNN, KNN * KNN), np.float32)
for _a in range(KNN):
    for _b in range(KNN):
        _E1_np[_a, _a * KNN + _b] = 1.0
        _E2_np[_b, _a * KNN + _b] = 1.0
_R_np = np.zeros((HEAD * KNN, HEAD * KNN * 16), np.float32)
for _j in range(HEAD * KNN):
    _R_np[_j, _j * 16:(_j + 1) * 16] = 1.0


NEG_INF = np.float32(-np.inf)
BIG_I32 = np.int32(1 << 30)


def _topk16(s, lane):
    vals, lanes = [], []
    for _ in range(KNN):
        mx = jnp.max(s, axis=1, keepdims=True)
        pick = jnp.min(jnp.where(s == mx, lane, BIG_I32), axis=1, keepdims=True)
        s = jnp.where(lane == pick, NEG_INF, s)
        vals.append(mx)
        lanes.append(pick)
    return vals, lanes


def _score_kernel(x_ref, wq_ref, keys_ref, e1_ref, e2_ref, r_ref, wsw_ref,
                  idx_ref, wrep_ref, gate_ref):
    t = x_ref.shape[0]
    x = x_ref[...]
    g = lax.dot_general(x, wsw_ref[...], (((1,), (1,)), ((), ())),
                        preferred_element_type=jnp.float32)
    gate_ref[...] = g / (1.0 + jnp.exp(-g))
    q = lax.dot_general(x, wq_ref[...], (((1,), (1,)), ((), ())),
                        preferred_element_type=jnp.float32)
    lane = lax.broadcasted_iota(jnp.int32, (t, KNUM), 1)
    iota16 = lax.broadcasted_iota(jnp.int32, (t, KNN), 1)
    w_heads = []
    for h in range(HEAD):
        svs, ifs = [], []
        for half in range(2):
            c = h * 2 + half
            qc = q[:, c * KDIM:(c + 1) * KDIM]
            m = jnp.mean(qc, axis=1, keepdims=True)
            d = qc - m
            v = jnp.mean(d * d, axis=1, keepdims=True)
            qn = d * lax.rsqrt(v + 1e-5)
            s = lax.dot_general(qn, keys_ref[c], (((1,), (1,)), ((), ())),
                                preferred_element_type=jnp.float32)
            vals, lanes = _topk16(s, lane)
            svs.append(jnp.concatenate(vals, axis=1))
            ifs.append(jnp.concatenate(lanes, axis=1).astype(jnp.float32))
        cand = (jnp.dot(svs[0], e1_ref[...], preferred_element_type=jnp.float32, precision=lax.Precision.HIGHEST)
                + jnp.dot(svs[1], e2_ref[...], preferred_element_type=jnp.float32, precision=lax.Precision.HIGHEST))
        sc_cols, vi_cols = [], []
        sref = cand
        for _ in range(KNN):
            mx = jnp.max(sref, axis=1, keepdims=True)
            j = jnp.min(jnp.where(sref == mx, lane, BIG_I32), axis=1, keepdims=True)
            sref = jnp.where(lane == j, NEG_INF, sref)
            a = lax.shift_right_arithmetic(j, 4)
            b = j & np.int32(15)
            i1k = jnp.sum(jnp.where(iota16 == a, ifs[0], 0.0), axis=1, keepdims=True)
            i2k = jnp.sum(jnp.where(iota16 == b, ifs[1], 0.0), axis=1, keepdims=True)
            vi_cols.append(i1k * np.float32(KNUM) + i2k)
            sc_cols.append(mx)
        sc = jnp.concatenate(sc_cols, axis=1)
        vidx = jnp.concatenate(vi_cols, axis=1)
        e = jnp.exp(sc - jnp.max(sc, axis=1, keepdims=True))
        w = e / jnp.sum(e, axis=1, keepdims=True)
        idx_ref[:, h * KNN:(h + 1) * KNN] = vidx.astype(jnp.int32)
        w_heads.append(w)
    wfull = jnp.concatenate(w_heads, axis=1)
    wrep_ref[...] = jnp.dot(wfull, r_ref[...],
                            preferred_element_type=jnp.float32, precision=lax.Precision.HIGHEST)


def _out_kernel(gate_ref, mem_ref, wv_ref, o_ref):
    hgate = mem_ref[...] * gate_ref[...]
    o_ref[...] = lax.dot_general(hgate, wv_ref[...], (((1,), (1,)), ((), ())),
                                 preferred_element_type=jnp.float32)


def _scores_tc(x, Wq, keys4, e1, e2, rrep, Wsw):
    grid = (BS // TILE,)
    return pl.pallas_call(
        _score_kernel,
        grid=grid,
        in_specs=[
            pl.BlockSpec((TILE, HIDDEN), lambda i: (i, 0)),
            pl.BlockSpec((KDIM * 2 * HEAD, HIDDEN), lambda i: (0, 0)),
            pl.BlockSpec((4, KNUM, KDIM), lambda i: (0, 0, 0)),
            pl.BlockSpec((KNN, KNN * KNN), lambda i: (0, 0)),
            pl.BlockSpec((KNN, KNN * KNN), lambda i: (0, 0)),
            pl.BlockSpec((HEAD * KNN, HEAD * KNN * 16), lambda i: (0, 0)),
            pl.BlockSpec((VDIM, HIDDEN), lambda i: (0, 0)),
        ],
        out_specs=[
            pl.BlockSpec((TILE, HEAD * KNN), lambda i: (i, 0)),
            pl.BlockSpec((TILE, HEAD * KNN * 16), lambda i: (i, 0)),
            pl.BlockSpec((TILE, VDIM), lambda i: (i, 0)),
        ],
        out_shape=[
            jax.ShapeDtypeStruct((BS, HEAD * KNN), jnp.int32),
            jax.ShapeDtypeStruct((BS, HEAD * KNN * 16), jnp.float32),
            jax.ShapeDtypeStruct((BS, VDIM), jnp.float32),
        ],
    )(x, Wq, keys4, e1, e2, rrep, Wsw)


def _output_tc(gate, mem_out, Wv):
    grid = (BS // TILE,)
    return pl.pallas_call(
        _out_kernel,
        grid=grid,
        in_specs=[
            pl.BlockSpec((TILE, VDIM), lambda i: (i, 0)),
            pl.BlockSpec((TILE, VDIM), lambda i: (i, 0)),
            pl.BlockSpec((HIDDEN, VDIM), lambda i: (0, 0)),
        ],
        out_specs=pl.BlockSpec((TILE, HIDDEN), lambda i: (i, 0)),
        out_shape=jax.ShapeDtypeStruct((BS, HIDDEN), jnp.float32),
    )(gate, mem_out, Wv)


_NC, _NS = 2, 16
_NW = _NC * _NS
_TPW = BS // _NW
_K = HEAD * KNN
_NG = VDIM // 16


def _sc_gather(indices, wrep, values):
    mesh = plsc.VectorSubcoreMesh(core_axis_name="c", subcore_axis_name="s")

    @functools.partial(
        pl.kernel,
        mesh=mesh,
        out_type=jax.ShapeDtypeStruct((BS, VDIM), jnp.float32),
        scratch_types=[
            pltpu.VMEM((_TPW, _K), jnp.int32),
            pltpu.VMEM((_TPW, _K * 16), jnp.float32),
            pltpu.VMEM((2, _K, VDIM), jnp.float32),
            pltpu.VMEM((VDIM,), jnp.float32),
            pltpu.SemaphoreType.DMA,
            pltpu.SemaphoreType.DMA,
        ],
    )
    def gather_kernel(idx_hbm, w_hbm, values_hbm, out_hbm,
                      idx_v, w_v, rows_v, acc_v, sem0, sem1):
        wid = lax.axis_index("s") * _NC + lax.axis_index("c")
        base = wid * _TPW
        pltpu.sync_copy(idx_hbm.at[pl.ds(base, _TPW)], idx_v)
        pltpu.sync_copy(w_hbm.at[pl.ds(base, _TPW)], w_v)

        def start_gather(tok, buf, sem):
            pltpu.async_copy(values_hbm.at[idx_v.at[tok]], rows_v.at[buf], sem)

        def wait(src_tok, buf, sem):
            pltpu.make_async_copy(values_hbm.at[idx_v.at[src_tok]],
                                  rows_v.at[buf], sem).wait()

        def compute(tok, buf):
            w_regs = [w_v[tok, pl.ds(r * 16, 16)] for r in range(_K)]

            def gbody(g, carry):
                o = g * 16
                accs = [rows_v[buf, r, pl.ds(o, 16)] * w_regs[r]
                        for r in range(8)]
                for r in range(8, _K):
                    c = r % 8
                    accs[c] = accs[c] + rows_v[buf, r, pl.ds(o, 16)] * w_regs[r]
                acc_v[pl.ds(o, 16)] = ((accs[0] + accs[1]) + (accs[2] + accs[3])
                                       + ((accs[4] + accs[5]) + (accs[6] + accs[7])))
                return carry

            lax.fori_loop(0, _NG, gbody, 0, unroll=2)
            pltpu.sync_copy(acc_v, out_hbm.at[base + tok])

        start_gather(0, 0, sem0)

        def body(t2, carry):
            t0 = t2 * 2
            t1 = t0 + 1
            start_gather(t1, 1, sem1)
            wait(t0, 0, sem0)
            compute(t0, 0)

            @pl.when(t2 < _TPW // 2 - 1)
            def _():
                start_gather(t0 + 2, 0, sem0)

            wait(t1, 1, sem1)
            compute(t1, 1)
            return carry

        lax.fori_loop(0, _TPW // 2, body, 0)

    return gather_kernel(indices, wrep, values)


def kernel(hidden_state, Wq, keys, Wv, Wsw, values_for_look_up):
    prefix = hidden_state.shape[:-1]
    x = hidden_state.reshape(BS, HIDDEN)
    keys4 = keys.reshape(HEAD * 2, KNUM, KDIM)
    e1 = jnp.asarray(_E1_np)
    e2 = jnp.asarray(_E2_np)
    indices, wrep, gate = _scores_tc(x, Wq, keys4, e1, e2, jnp.asarray(_R_np), Wsw)
    mem_out = _sc_gather(indices, wrep, values_for_look_up)
    out = _output_tc(gate, mem_out, Wv)
    return out.reshape(prefix + (HIDDEN,))


# speedup vs baseline: 2.6209x; 1.3322x over previous
import functools

import jax
import jax.numpy as jnp
import numpy as np
from jax import lax
from jax.experimental import pallas as pl
from jax.experimental.pallas import tpu as pltpu
from jax.experimental.pallas import tpu_sc as plsc

HIDDEN = 1024
KDIM = 256
KNUM = 256
VDIM = 1024
KNN = 16
HEAD = 2

BS = 2048
TILE = 256
INT_MIN = np.int32(-2147483648)
MASK_LOW = np.int32(-256)
LOW = np.int32(255)

_E1_np = np.zeros((KNN, KNN * KNN), np.float32)
_E2_np = np.zeros((KNN, KNN * KNN), np.float32)
for _a in range(KNN):
    for _b in range(KNN):
        _E1_np[_a, _a * KNN + _b] = 1.0
        _E2_np[_b, _a * KNN + _b] = 1.0
_R_np = np.zeros((HEAD * KNN, HEAD * KNN * 16), np.float32)
for _j in range(HEAD * KNN):
    _R_np[_j, _j * 16:(_j + 1) * 16] = 1.0


NEG_INF = np.float32(-np.inf)
BIG_I32 = np.int32(1 << 30)


def _topk16_t(s, lane):
    vals, lanes = [], []
    for _ in range(KNN):
        mx = jnp.max(s, axis=0, keepdims=True)
        pick = jnp.min(jnp.where(s == mx, lane, BIG_I32), axis=0, keepdims=True)
        s = jnp.where(lane == pick, NEG_INF, s)
        vals.append(mx)
        lanes.append(pick)
    return vals, lanes


def _score_kernel(x_ref, wq_ref, keys_ref, e1t_ref, e2t_ref, rt_ref, wsw_ref,
                  idx_ref, wrep_ref, gate_ref):
    t = x_ref.shape[0]
    x = x_ref[...]
    g = lax.dot_general(x, wsw_ref[...], (((1,), (1,)), ((), ())),
                        preferred_element_type=jnp.float32)
    gate_ref[...] = g / (1.0 + jnp.exp(-g))
    q = lax.dot_general(x, wq_ref[...], (((1,), (1,)), ((), ())),
                        preferred_element_type=jnp.float32)
    lane = lax.broadcasted_iota(jnp.int32, (KNUM, t), 0)
    iota16 = lax.broadcasted_iota(jnp.int32, (KNN, t), 0)
    w_heads = []
    for h in range(HEAD):
        svs, ifs = [], []
        for half in range(2):
            c = h * 2 + half
            qc = q[:, c * KDIM:(c + 1) * KDIM]
            m = jnp.mean(qc, axis=1, keepdims=True)
            d = qc - m
            v = jnp.mean(d * d, axis=1, keepdims=True)
            qn = d * lax.rsqrt(v + 1e-5)
            s = lax.dot_general(keys_ref[c], qn, (((1,), (1,)), ((), ())),
                                preferred_element_type=jnp.float32)
            vals, lanes = _topk16_t(s, lane)
            svs.append(jnp.concatenate(vals, axis=0))
            ifs.append(jnp.concatenate(lanes, axis=0).astype(jnp.float32))
        cand = (jnp.dot(e1t_ref[...], svs[0], preferred_element_type=jnp.float32, precision=lax.Precision.HIGHEST)
                + jnp.dot(e2t_ref[...], svs[1], preferred_element_type=jnp.float32, precision=lax.Precision.HIGHEST))
        sc_cols, vi_cols = [], []
        sref = cand
        for _ in range(KNN):
            mx = jnp.max(sref, axis=0, keepdims=True)
            j = jnp.min(jnp.where(sref == mx, lane, BIG_I32), axis=0, keepdims=True)
            sref = jnp.where(lane == j, NEG_INF, sref)
            a = lax.shift_right_arithmetic(j, 4)
            b = j & np.int32(15)
            i1k = jnp.sum(jnp.where(iota16 == a, ifs[0], 0.0), axis=0, keepdims=True)
            i2k = jnp.sum(jnp.where(iota16 == b, ifs[1], 0.0), axis=0, keepdims=True)
            vi_cols.append(i1k * np.float32(KNUM) + i2k)
            sc_cols.append(mx)
        sc = jnp.concatenate(sc_cols, axis=0)
        vidx = jnp.concatenate(vi_cols, axis=0)
        e = jnp.exp(sc - jnp.max(sc, axis=0, keepdims=True))
        w = e / jnp.sum(e, axis=0, keepdims=True)
        idx_ref[h * KNN:(h + 1) * KNN, :] = vidx.astype(jnp.int32)
        w_heads.append(w)
    wfull = jnp.concatenate(w_heads, axis=0)
    wrep_ref[...] = jnp.dot(rt_ref[...], wfull,
                            preferred_element_type=jnp.float32, precision=lax.Precision.HIGHEST)


def _out_kernel(gate_ref, mem_ref, wv_ref, o_ref):
    hgate = mem_ref[...] * gate_ref[...]
    o_ref[...] = lax.dot_general(hgate, wv_ref[...], (((1,), (1,)), ((), ())),
                                 preferred_element_type=jnp.float32)


def _scores_tc(x, Wq, keys4, e1, e2, rrep, Wsw):
    grid = (BS // TILE,)
    return pl.pallas_call(
        _score_kernel,
        grid=grid,
        in_specs=[
            pl.BlockSpec((TILE, HIDDEN), lambda i: (i, 0)),
            pl.BlockSpec((KDIM * 2 * HEAD, HIDDEN), lambda i: (0, 0)),
            pl.BlockSpec((4, KNUM, KDIM), lambda i: (0, 0, 0)),
            pl.BlockSpec((KNN * KNN, KNN), lambda i: (0, 0)),
            pl.BlockSpec((KNN * KNN, KNN), lambda i: (0, 0)),
            pl.BlockSpec((HEAD * KNN * 16, HEAD * KNN), lambda i: (0, 0)),
            pl.BlockSpec((VDIM, HIDDEN), lambda i: (0, 0)),
        ],
        out_specs=[
            pl.BlockSpec((HEAD * KNN, TILE), lambda i: (0, i)),
            pl.BlockSpec((HEAD * KNN * 16, TILE), lambda i: (0, i)),
            pl.BlockSpec((TILE, VDIM), lambda i: (i, 0)),
        ],
        out_shape=[
            jax.ShapeDtypeStruct((HEAD * KNN, BS), jnp.int32),
            jax.ShapeDtypeStruct((HEAD * KNN * 16, BS), jnp.float32),
            jax.ShapeDtypeStruct((BS, VDIM), jnp.float32),
        ],
    )(x, Wq, keys4, e1, e2, rrep, Wsw)


def _output_tc(gate, mem_out, Wv):
    grid = (BS // TILE,)
    return pl.pallas_call(
        _out_kernel,
        grid=grid,
        in_specs=[
            pl.BlockSpec((TILE, VDIM), lambda i: (i, 0)),
            pl.BlockSpec((TILE, VDIM), lambda i: (i, 0)),
            pl.BlockSpec((HIDDEN, VDIM), lambda i: (0, 0)),
        ],
        out_specs=pl.BlockSpec((TILE, HIDDEN), lambda i: (i, 0)),
        out_shape=jax.ShapeDtypeStruct((BS, HIDDEN), jnp.float32),
    )(gate, mem_out, Wv)


_NC, _NS = 2, 16
_NW = _NC * _NS
_TPW = BS // _NW
_K = HEAD * KNN
_NG = VDIM // 16


def _sc_gather(indices, wrep, values):
    mesh = plsc.VectorSubcoreMesh(core_axis_name="c", subcore_axis_name="s")

    @functools.partial(
        pl.kernel,
        mesh=mesh,
        out_type=jax.ShapeDtypeStruct((BS, VDIM), jnp.float32),
        scratch_types=[
            pltpu.VMEM((_TPW, _K), jnp.int32),
            pltpu.VMEM((_TPW, _K * 16), jnp.float32),
            pltpu.VMEM((2, _K, VDIM), jnp.float32),
            pltpu.VMEM((VDIM,), jnp.float32),
            pltpu.SemaphoreType.DMA,
            pltpu.SemaphoreType.DMA,
        ],
    )
    def gather_kernel(idx_hbm, w_hbm, values_hbm, out_hbm,
                      idx_v, w_v, rows_v, acc_v, sem0, sem1):
        wid = lax.axis_index("s") * _NC + lax.axis_index("c")
        base = wid * _TPW
        pltpu.sync_copy(idx_hbm.at[pl.ds(base, _TPW)], idx_v)
        pltpu.sync_copy(w_hbm.at[pl.ds(base, _TPW)], w_v)

        def start_gather(tok, buf, sem):
            pltpu.async_copy(values_hbm.at[idx_v.at[tok]], rows_v.at[buf], sem)

        def wait(src_tok, buf, sem):
            pltpu.make_async_copy(values_hbm.at[idx_v.at[src_tok]],
                                  rows_v.at[buf], sem).wait()

        def compute(tok, buf):
            w_regs = [w_v[tok, pl.ds(r * 16, 16)] for r in range(_K)]

            def gbody(g, carry):
                o = g * 16
                accs = [rows_v[buf, r, pl.ds(o, 16)] * w_regs[r]
                        for r in range(8)]
                for r in range(8, _K):
                    c = r % 8
                    accs[c] = accs[c] + rows_v[buf, r, pl.ds(o, 16)] * w_regs[r]
                acc_v[pl.ds(o, 16)] = ((accs[0] + accs[1]) + (accs[2] + accs[3])
                                       + ((accs[4] + accs[5]) + (accs[6] + accs[7])))
                return carry

            lax.fori_loop(0, _NG, gbody, 0, unroll=2)
            pltpu.sync_copy(acc_v, out_hbm.at[base + tok])

        start_gather(0, 0, sem0)

        def body(t2, carry):
            t0 = t2 * 2
            t1 = t0 + 1
            start_gather(t1, 1, sem1)
            wait(t0, 0, sem0)
            compute(t0, 0)

            @pl.when(t2 < _TPW // 2 - 1)
            def _():
                start_gather(t0 + 2, 0, sem0)

            wait(t1, 1, sem1)
            compute(t1, 1)
            return carry

        lax.fori_loop(0, _TPW // 2, body, 0)

    return gather_kernel(indices, wrep, values)


def kernel(hidden_state, Wq, keys, Wv, Wsw, values_for_look_up):
    prefix = hidden_state.shape[:-1]
    x = hidden_state.reshape(BS, HIDDEN)
    keys4 = keys.reshape(HEAD * 2, KNUM, KDIM)
    e1 = jnp.asarray(_E1_np.T)
    e2 = jnp.asarray(_E2_np.T)
    indices_t, wrep_t, gate = _scores_tc(x, Wq, keys4, e1, e2,
                                         jnp.asarray(_R_np.T), Wsw)
    mem_out = _sc_gather(indices_t.T, wrep_t.T, values_for_look_up)
    out = _output_tc(gate, mem_out, Wv)
    return out.reshape(prefix + (HIDDEN,))


# speedup vs baseline: 2.6666x; 1.0174x over previous
import functools

import jax
import jax.numpy as jnp
import numpy as np
from jax import lax
from jax.experimental import pallas as pl
from jax.experimental.pallas import tpu as pltpu
from jax.experimental.pallas import tpu_sc as plsc

HIDDEN = 1024
KDIM = 256
KNUM = 256
VDIM = 1024
KNN = 16
HEAD = 2

BS = 2048
TILE = 256
INT_MIN = np.int32(-2147483648)
MASK_LOW = np.int32(-256)
LOW = np.int32(255)

_E1_np = np.zeros((KNN, KNN * KNN), np.float32)
_E2_np = np.zeros((KNN, KNN * KNN), np.float32)
for _a in range(KNN):
    for _b in range(KNN):
        _E1_np[_a, _a * KNN + _b] = 1.0
        _E2_np[_b, _a * KNN + _b] = 1.0
_R_np = np.zeros((HEAD * KNN, HEAD * KNN * 16), np.float32)
for _j in range(HEAD * KNN):
    _R_np[_j, _j * 16:(_j + 1) * 16] = 1.0


NEG_INF = np.float32(-np.inf)
BIG_I32 = np.int32(1 << 30)


def _topk16_t(s, lane):
    vals, lanes = [], []
    for _ in range(KNN):
        mx = jnp.max(s, axis=0, keepdims=True)
        pick = jnp.min(jnp.where(s == mx, lane, BIG_I32), axis=0, keepdims=True)
        s = jnp.where(lane == pick, NEG_INF, s)
        vals.append(mx)
        lanes.append(pick)
    return vals, lanes


def _gate_kernel(x_ref, wsw_ref, gate_ref):
    g = lax.dot_general(x_ref[...], wsw_ref[...], (((1,), (1,)), ((), ())),
                        preferred_element_type=jnp.float32)
    gate_ref[...] = g / (1.0 + jnp.exp(-g))


def _score_kernel(x_ref, wq_ref, keys_ref, e1t_ref, e2t_ref, rt_ref,
                  idx_ref, wrep_ref):
    t = x_ref.shape[0]
    x = x_ref[...]
    q = lax.dot_general(x, wq_ref[...], (((1,), (1,)), ((), ())),
                        preferred_element_type=jnp.float32)
    lane = lax.broadcasted_iota(jnp.int32, (KNUM, t), 0)
    iota16 = lax.broadcasted_iota(jnp.int32, (KNN, t), 0)
    w_heads = []
    for h in range(HEAD):
        svs, ifs = [], []
        for half in range(2):
            c = h * 2 + half
            qc = q[:, c * KDIM:(c + 1) * KDIM]
            m = jnp.mean(qc, axis=1, keepdims=True)
            d = qc - m
            v = jnp.mean(d * d, axis=1, keepdims=True)
            qn = d * lax.rsqrt(v + 1e-5)
            s = lax.dot_general(keys_ref[c], qn, (((1,), (1,)), ((), ())),
                                preferred_element_type=jnp.float32)
            vals, lanes = _topk16_t(s, lane)
            svs.append(jnp.concatenate(vals, axis=0))
            ifs.append(jnp.concatenate(lanes, axis=0).astype(jnp.float32))
        cand = (jnp.dot(e1t_ref[...], svs[0], preferred_element_type=jnp.float32, precision=lax.Precision.HIGHEST)
                + jnp.dot(e2t_ref[...], svs[1], preferred_element_type=jnp.float32, precision=lax.Precision.HIGHEST))
        sc_cols, vi_cols = [], []
        sref = cand
        for _ in range(KNN):
            mx = jnp.max(sref, axis=0, keepdims=True)
            j = jnp.min(jnp.where(sref == mx, lane, BIG_I32), axis=0, keepdims=True)
            sref = jnp.where(lane == j, NEG_INF, sref)
            a = lax.shift_right_arithmetic(j, 4)
            b = j & np.int32(15)
            i1k = jnp.sum(jnp.where(iota16 == a, ifs[0], 0.0), axis=0, keepdims=True)
            i2k = jnp.sum(jnp.where(iota16 == b, ifs[1], 0.0), axis=0, keepdims=True)
            vi_cols.append(i1k * np.float32(KNUM) + i2k)
            sc_cols.append(mx)
        sc = jnp.concatenate(sc_cols, axis=0)
        vidx = jnp.concatenate(vi_cols, axis=0)
        e = jnp.exp(sc - jnp.max(sc, axis=0, keepdims=True))
        w = e / jnp.sum(e, axis=0, keepdims=True)
        idx_ref[h * KNN:(h + 1) * KNN, :] = vidx.astype(jnp.int32)
        w_heads.append(w)
    wfull = jnp.concatenate(w_heads, axis=0)
    wrep_ref[...] = jnp.dot(rt_ref[...], wfull,
                            preferred_element_type=jnp.float32, precision=lax.Precision.HIGHEST)


def _out_kernel(gate_ref, mem_ref, wv_ref, o_ref):
    hgate = mem_ref[...] * gate_ref[...]
    o_ref[...] = lax.dot_general(hgate, wv_ref[...], (((1,), (1,)), ((), ())),
                                 preferred_element_type=jnp.float32)


def _scores_tc(x, Wq, keys4, e1, e2, rrep):
    grid = (BS // TILE,)
    return pl.pallas_call(
        _score_kernel,
        grid=grid,
        in_specs=[
            pl.BlockSpec((TILE, HIDDEN), lambda i: (i, 0)),
            pl.BlockSpec((KDIM * 2 * HEAD, HIDDEN), lambda i: (0, 0)),
            pl.BlockSpec((4, KNUM, KDIM), lambda i: (0, 0, 0)),
            pl.BlockSpec((KNN * KNN, KNN), lambda i: (0, 0)),
            pl.BlockSpec((KNN * KNN, KNN), lambda i: (0, 0)),
            pl.BlockSpec((HEAD * KNN * 16, HEAD * KNN), lambda i: (0, 0)),
        ],
        out_specs=[
            pl.BlockSpec((HEAD * KNN, TILE), lambda i: (0, i)),
            pl.BlockSpec((HEAD * KNN * 16, TILE), lambda i: (0, i)),
        ],
        out_shape=[
            jax.ShapeDtypeStruct((HEAD * KNN, BS), jnp.int32),
            jax.ShapeDtypeStruct((HEAD * KNN * 16, BS), jnp.float32),
        ],
    )(x, Wq, keys4, e1, e2, rrep)


def _gate_tc(x, Wsw):
    grid = (BS // TILE,)
    return pl.pallas_call(
        _gate_kernel,
        grid=grid,
        in_specs=[
            pl.BlockSpec((TILE, HIDDEN), lambda i: (i, 0)),
            pl.BlockSpec((VDIM, HIDDEN), lambda i: (0, 0)),
        ],
        out_specs=pl.BlockSpec((TILE, VDIM), lambda i: (i, 0)),
        out_shape=jax.ShapeDtypeStruct((BS, VDIM), jnp.float32),
    )(x, Wsw)


def _output_tc(gate, mem_out, Wv):
    grid = (BS // TILE,)
    return pl.pallas_call(
        _out_kernel,
        grid=grid,
        in_specs=[
            pl.BlockSpec((TILE, VDIM), lambda i: (i, 0)),
            pl.BlockSpec((TILE, VDIM), lambda i: (i, 0)),
            pl.BlockSpec((HIDDEN, VDIM), lambda i: (0, 0)),
        ],
        out_specs=pl.BlockSpec((TILE, HIDDEN), lambda i: (i, 0)),
        out_shape=jax.ShapeDtypeStruct((BS, HIDDEN), jnp.float32),
    )(gate, mem_out, Wv)


_NC, _NS = 2, 16
_NW = _NC * _NS
_TPW = BS // _NW
_K = HEAD * KNN
_NG = VDIM // 16


def _sc_gather(indices, wrep, values):
    mesh = plsc.VectorSubcoreMesh(core_axis_name="c", subcore_axis_name="s")

    @functools.partial(
        pl.kernel,
        mesh=mesh,
        out_type=jax.ShapeDtypeStruct((BS, VDIM), jnp.float32),
        scratch_types=[
            pltpu.VMEM((_TPW, _K), jnp.int32),
            pltpu.VMEM((_TPW, _K * 16), jnp.float32),
            pltpu.VMEM((2, _K, VDIM), jnp.float32),
            pltpu.VMEM((VDIM,), jnp.float32),
            pltpu.SemaphoreType.DMA,
            pltpu.SemaphoreType.DMA,
        ],
    )
    def gather_kernel(idx_hbm, w_hbm, values_hbm, out_hbm,
                      idx_v, w_v, rows_v, acc_v, sem0, sem1):
        wid = lax.axis_index("s") * _NC + lax.axis_index("c")
        base = wid * _TPW
        pltpu.sync_copy(idx_hbm.at[pl.ds(base, _TPW)], idx_v)
        pltpu.sync_copy(w_hbm.at[pl.ds(base, _TPW)], w_v)

        def start_gather(tok, buf, sem):
            pltpu.async_copy(values_hbm.at[idx_v.at[tok]], rows_v.at[buf], sem)

        def wait(src_tok, buf, sem):
            pltpu.make_async_copy(values_hbm.at[idx_v.at[src_tok]],
                                  rows_v.at[buf], sem).wait()

        def compute(tok, buf):
            w_regs = [w_v[tok, pl.ds(r * 16, 16)] for r in range(_K)]

            def gbody(g, carry):
                o = g * 16
                accs = [rows_v[buf, r, pl.ds(o, 16)] * w_regs[r]
                        for r in range(8)]
                for r in range(8, _K):
                    c = r % 8
                    accs[c] = accs[c] + rows_v[buf, r, pl.ds(o, 16)] * w_regs[r]
                acc_v[pl.ds(o, 16)] = ((accs[0] + accs[1]) + (accs[2] + accs[3])
                                       + ((accs[4] + accs[5]) + (accs[6] + accs[7])))
                return carry

            lax.fori_loop(0, _NG, gbody, 0, unroll=2)
            pltpu.sync_copy(acc_v, out_hbm.at[base + tok])

        start_gather(0, 0, sem0)

        def body(t2, carry):
            t0 = t2 * 2
            t1 = t0 + 1
            start_gather(t1, 1, sem1)
            wait(t0, 0, sem0)
            compute(t0, 0)

            @pl.when(t2 < _TPW // 2 - 1)
            def _():
                start_gather(t0 + 2, 0, sem0)

            wait(t1, 1, sem1)
            compute(t1, 1)
            return carry

        lax.fori_loop(0, _TPW // 2, body, 0)

    return gather_kernel(indices, wrep, values)


def kernel(hidden_state, Wq, keys, Wv, Wsw, values_for_look_up):
    prefix = hidden_state.shape[:-1]
    x = hidden_state.reshape(BS, HIDDEN)
    keys4 = keys.reshape(HEAD * 2, KNUM, KDIM)
    e1 = jnp.asarray(_E1_np.T)
    e2 = jnp.asarray(_E2_np.T)
    indices_t, wrep_t = _scores_tc(x, Wq, keys4, e1, e2, jnp.asarray(_R_np.T))
    mem_out = _sc_gather(indices_t.T, wrep_t.T, values_for_look_up)
    gate = _gate_tc(x, Wsw)
    out = _output_tc(gate, mem_out, Wv)
    return out.reshape(prefix + (HIDDEN,))


# speedup vs baseline: 2.7432x; 1.0287x over previous
import functools

import jax
import jax.numpy as jnp
import numpy as np
from jax import lax
from jax.experimental import pallas as pl
from jax.experimental.pallas import tpu as pltpu
from jax.experimental.pallas import tpu_sc as plsc

HIDDEN = 1024
KDIM = 256
KNUM = 256
VDIM = 1024
KNN = 16
HEAD = 2

BS = 2048
TILE = 256
INT_MIN = np.int32(-2147483648)
MASK_LOW = np.int32(-256)
LOW = np.int32(255)

_E1_np = np.zeros((KNN, KNN * KNN), np.float32)
_E2_np = np.zeros((KNN, KNN * KNN), np.float32)
for _a in range(KNN):
    for _b in range(KNN):
        _E1_np[_a, _a * KNN + _b] = 1.0
        _E2_np[_b, _a * KNN + _b] = 1.0
_R_np = np.zeros((HEAD * KNN, HEAD * KNN * 16), np.float32)
for _j in range(HEAD * KNN):
    _R_np[_j, _j * 16:(_j + 1) * 16] = 1.0


NEG_INF = np.float32(-np.inf)
BIG_I32 = np.int32(1 << 30)


def _topk16_t(s, lane):
    vals, lanes = [], []
    for _ in range(KNN):
        mx = jnp.max(s, axis=0, keepdims=True)
        pick = jnp.min(jnp.where(s == mx, lane, BIG_I32), axis=0, keepdims=True)
        s = jnp.where(lane == pick, NEG_INF, s)
        vals.append(mx)
        lanes.append(pick)
    return vals, lanes


def _gate_kernel(x_ref, wsw_ref, gate_ref):
    g = lax.dot_general(x_ref[...], wsw_ref[...], (((1,), (1,)), ((), ())),
                        preferred_element_type=jnp.float32)
    gate_ref[...] = g / (1.0 + jnp.exp(-g))


def _score_kernel(x_ref, wq_ref, keys_ref, e1t_ref, e2t_ref, r_ref, eye_ref,
                  idx_ref, wrep_ref):
    t = x_ref.shape[0]
    x = x_ref[...]
    q = lax.dot_general(x, wq_ref[...], (((1,), (1,)), ((), ())),
                        preferred_element_type=jnp.float32)
    lane = lax.broadcasted_iota(jnp.int32, (KNUM, t), 0)
    iota16 = lax.broadcasted_iota(jnp.int32, (KNN, t), 0)
    w_heads, vi_heads = [], []
    for h in range(HEAD):
        svs, ifs = [], []
        for half in range(2):
            c = h * 2 + half
            qc = q[:, c * KDIM:(c + 1) * KDIM]
            m = jnp.mean(qc, axis=1, keepdims=True)
            d = qc - m
            v = jnp.mean(d * d, axis=1, keepdims=True)
            qn = d * lax.rsqrt(v + 1e-5)
            s = lax.dot_general(keys_ref[c], qn, (((1,), (1,)), ((), ())),
                                preferred_element_type=jnp.float32)
            vals, lanes = _topk16_t(s, lane)
            svs.append(jnp.concatenate(vals, axis=0))
            ifs.append(jnp.concatenate(lanes, axis=0).astype(jnp.float32))
        cand = (jnp.dot(e1t_ref[...], svs[0], preferred_element_type=jnp.float32, precision=lax.Precision.HIGHEST)
                + jnp.dot(e2t_ref[...], svs[1], preferred_element_type=jnp.float32, precision=lax.Precision.HIGHEST))
        sc_cols, vi_cols = [], []
        sref = cand
        for _ in range(KNN):
            mx = jnp.max(sref, axis=0, keepdims=True)
            j = jnp.min(jnp.where(sref == mx, lane, BIG_I32), axis=0, keepdims=True)
            sref = jnp.where(lane == j, NEG_INF, sref)
            a = lax.shift_right_arithmetic(j, 4)
            b = j & np.int32(15)
            i1k = jnp.sum(jnp.where(iota16 == a, ifs[0], 0.0), axis=0, keepdims=True)
            i2k = jnp.sum(jnp.where(iota16 == b, ifs[1], 0.0), axis=0, keepdims=True)
            vi_cols.append(i1k * np.float32(KNUM) + i2k)
            sc_cols.append(mx)
        sc = jnp.concatenate(sc_cols, axis=0)
        vidx = jnp.concatenate(vi_cols, axis=0)
        e = jnp.exp(sc - jnp.max(sc, axis=0, keepdims=True))
        w = e / jnp.sum(e, axis=0, keepdims=True)
        vi_heads.append(vidx)
        w_heads.append(w)
    wfull = jnp.concatenate(w_heads, axis=0)
    viall = jnp.concatenate(vi_heads, axis=0)
    wrep_ref[...] = lax.dot_general(wfull, r_ref[...], (((0,), (0,)), ((), ())),
                                    preferred_element_type=jnp.float32,
                                    precision=lax.Precision.HIGHEST)
    idxf = lax.dot_general(viall, eye_ref[...], (((0,), (0,)), ((), ())),
                           preferred_element_type=jnp.float32,
                           precision=lax.Precision.HIGHEST)
    idx_ref[...] = (idxf + 0.5).astype(jnp.int32)


def _out_kernel(gate_ref, mem_ref, wv_ref, o_ref):
    hgate = mem_ref[...] * gate_ref[...]
    o_ref[...] = lax.dot_general(hgate, wv_ref[...], (((1,), (1,)), ((), ())),
                                 preferred_element_type=jnp.float32)


def _scores_tc(x, Wq, keys4, e1, e2, rrep):
    grid = (BS // TILE,)
    return pl.pallas_call(
        _score_kernel,
        grid=grid,
        in_specs=[
            pl.BlockSpec((TILE, HIDDEN), lambda i: (i, 0)),
            pl.BlockSpec((KDIM * 2 * HEAD, HIDDEN), lambda i: (0, 0)),
            pl.BlockSpec((4, KNUM, KDIM), lambda i: (0, 0, 0)),
            pl.BlockSpec((KNN * KNN, KNN), lambda i: (0, 0)),
            pl.BlockSpec((KNN * KNN, KNN), lambda i: (0, 0)),
            pl.BlockSpec((HEAD * KNN, HEAD * KNN * 16), lambda i: (0, 0)),
            pl.BlockSpec((HEAD * KNN, HEAD * KNN), lambda i: (0, 0)),
        ],
        out_specs=[
            pl.BlockSpec((TILE, HEAD * KNN), lambda i: (i, 0)),
            pl.BlockSpec((TILE, HEAD * KNN * 16), lambda i: (i, 0)),
        ],
        out_shape=[
            jax.ShapeDtypeStruct((BS, HEAD * KNN), jnp.int32),
            jax.ShapeDtypeStruct((BS, HEAD * KNN * 16), jnp.float32),
        ],
    )(x, Wq, keys4, e1, e2, rrep, jnp.asarray(np.eye(HEAD * KNN, dtype=np.float32)))


def _gate_tc(x, Wsw):
    grid = (BS // TILE,)
    return pl.pallas_call(
        _gate_kernel,
        grid=grid,
        in_specs=[
            pl.BlockSpec((TILE, HIDDEN), lambda i: (i, 0)),
            pl.BlockSpec((VDIM, HIDDEN), lambda i: (0, 0)),
        ],
        out_specs=pl.BlockSpec((TILE, VDIM), lambda i: (i, 0)),
        out_shape=jax.ShapeDtypeStruct((BS, VDIM), jnp.float32),
    )(x, Wsw)


def _output_tc(gate, mem_out, Wv):
    grid = (BS // TILE,)
    return pl.pallas_call(
        _out_kernel,
        grid=grid,
        in_specs=[
            pl.BlockSpec((TILE, VDIM), lambda i: (i, 0)),
            pl.BlockSpec((TILE, VDIM), lambda i: (i, 0)),
            pl.BlockSpec((HIDDEN, VDIM), lambda i: (0, 0)),
        ],
        out_specs=pl.BlockSpec((TILE, HIDDEN), lambda i: (i, 0)),
        out_shape=jax.ShapeDtypeStruct((BS, HIDDEN), jnp.float32),
    )(gate, mem_out, Wv)


_NC, _NS = 2, 16
_NW = _NC * _NS
_TPW = BS // _NW
_K = HEAD * KNN
_NG = VDIM // 16


def _sc_gather(indices, wrep, values):
    mesh = plsc.VectorSubcoreMesh(core_axis_name="c", subcore_axis_name="s")

    @functools.partial(
        pl.kernel,
        mesh=mesh,
        out_type=jax.ShapeDtypeStruct((BS, VDIM), jnp.float32),
        scratch_types=[
            pltpu.VMEM((_TPW, _K), jnp.int32),
            pltpu.VMEM((_TPW, _K * 16), jnp.float32),
            pltpu.VMEM((2, _K, VDIM), jnp.float32),
            pltpu.VMEM((VDIM,), jnp.float32),
            pltpu.SemaphoreType.DMA,
            pltpu.SemaphoreType.DMA,
        ],
    )
    def gather_kernel(idx_hbm, w_hbm, values_hbm, out_hbm,
                      idx_v, w_v, rows_v, acc_v, sem0, sem1):
        wid = lax.axis_index("s") * _NC + lax.axis_index("c")
        base = wid * _TPW
        pltpu.sync_copy(idx_hbm.at[pl.ds(base, _TPW)], idx_v)
        pltpu.sync_copy(w_hbm.at[pl.ds(base, _TPW)], w_v)

        def start_gather(tok, buf, sem):
            pltpu.async_copy(values_hbm.at[idx_v.at[tok]], rows_v.at[buf], sem)

        def wait(src_tok, buf, sem):
            pltpu.make_async_copy(values_hbm.at[idx_v.at[src_tok]],
                                  rows_v.at[buf], sem).wait()

        def compute(tok, buf):
            w_regs = [w_v[tok, pl.ds(r * 16, 16)] for r in range(_K)]

            def gbody(g, carry):
                o = g * 16
                accs = [rows_v[buf, r, pl.ds(o, 16)] * w_regs[r]
                        for r in range(8)]
                for r in range(8, _K):
                    c = r % 8
                    accs[c] = accs[c] + rows_v[buf, r, pl.ds(o, 16)] * w_regs[r]
                acc_v[pl.ds(o, 16)] = ((accs[0] + accs[1]) + (accs[2] + accs[3])
                                       + ((accs[4] + accs[5]) + (accs[6] + accs[7])))
                return carry

            lax.fori_loop(0, _NG, gbody, 0, unroll=2)
            pltpu.sync_copy(acc_v, out_hbm.at[base + tok])

        start_gather(0, 0, sem0)

        def body(t2, carry):
            t0 = t2 * 2
            t1 = t0 + 1
            start_gather(t1, 1, sem1)
            wait(t0, 0, sem0)
            compute(t0, 0)

            @pl.when(t2 < _TPW // 2 - 1)
            def _():
                start_gather(t0 + 2, 0, sem0)

            wait(t1, 1, sem1)
            compute(t1, 1)
            return carry

        lax.fori_loop(0, _TPW // 2, body, 0)

    return gather_kernel(indices, wrep, values)


def kernel(hidden_state, Wq, keys, Wv, Wsw, values_for_look_up):
    prefix = hidden_state.shape[:-1]
    x = hidden_state.reshape(BS, HIDDEN)
    keys4 = keys.reshape(HEAD * 2, KNUM, KDIM)
    e1 = jnp.asarray(_E1_np.T)
    e2 = jnp.asarray(_E2_np.T)
    indices, wrep = _scores_tc(x, Wq, keys4, e1, e2, jnp.asarray(_R_np))
    mem_out = _sc_gather(indices, wrep, values_for_look_up)
    gate = _gate_tc(x, Wsw)
    out = _output_tc(gate, mem_out, Wv)
    return out.reshape(prefix + (HIDDEN,))
